# baseline (speedup 1.0000x reference)
import functools

import jax
import jax.numpy as jnp
from jax import lax
from jax.experimental import pallas as pl
from jax.experimental.pallas import tpu as pltpu
from jax.experimental.pallas import tpu_sc as plsc

N, E, D = 10000, 320000, 128
NC, NS = 2, 16
NW = NC * NS
EC = E // NW
K = 80
NB = EC // K
RPT = 624
RLAST = N - RPT * (NS - 1)
DEGW = 16
RB = 1000

_mesh = plsc.VectorSubcoreMesh(core_axis_name="c", subcore_axis_name="s")


def _rows_copy(s, copy):
    @pl.when(s < NS - 1)
    def _():
        copy(pl.multiple_of(s * RPT, 8), RPT)

    @pl.when(s == NS - 1)
    def _():
        copy((NS - 1) * RPT, RLAST)


@functools.partial(
    pl.kernel,
    out_type=jax.ShapeDtypeStruct((NC * N,), jnp.float32),
    mesh=_mesh,
    scratch_types=[
        pltpu.VMEM((EC,), jnp.int32),
        [pltpu.VMEM((K,), jnp.int32)] * 4,
        pltpu.VMEM((K,), jnp.float32),
        pltpu.VMEM((RLAST,), jnp.float32),
        pltpu.VMEM_SHARED((N,), jnp.float32),
        [pltpu.SemaphoreType.DMA] * 4,
    ],
)
def _deg_kernel(dst_hbm, out_hbm, dstbig, dv, ones_v, zbuf, acc, sems):
    c = lax.axis_index("c")
    s = lax.axis_index("s")
    t = s * NC + c
    zeros = jnp.zeros((16,), jnp.float32)
    for j in range(RLAST // 16):
        zbuf[pl.ds(j * 16, 16)] = zeros
    _rows_copy(s, lambda r0, nr: pltpu.sync_copy(
        zbuf.at[pl.ds(0, nr)], acc.at[pl.ds(r0, nr)]))
    ones = jnp.full((16,), 1.0, jnp.float32)
    for j in range(K // 16):
        ones_v[pl.ds(j * 16, 16)] = ones
    pltpu.sync_copy(dst_hbm.at[pl.ds(pl.multiple_of(t * EC, 8), EC)], dstbig)
    plsc.subcore_barrier()

    def fill(i, x):
        for j in range(K // 16):
            dv[x][pl.ds(j * 16, 16)] = dstbig[pl.ds(i * K + j * 16, 16)]

    def scat(x):
        pltpu.async_copy(ones_v, acc.at[dv[x]], sems[x], add=True)

    def wait_scat(x):
        pltpu.make_async_copy(ones_v, acc.at[dv[x]], sems[x]).wait()

    for i in range(4):
        fill(i, i)
        scat(i)

    def body(j, carry):
        i0 = 4 + 4 * j
        for u in range(4):
            wait_scat(u)
            fill(i0 + u, u)
            scat(u)
        return carry

    lax.fori_loop(0, (NB - 5) // 4, body, 0)
    wait_scat(0)
    fill(NB - 1, 0)
    scat(0)
    for x in (1, 2, 3, 0):
        wait_scat(x)
    plsc.subcore_barrier()

    def drain(r0, nr):
        pltpu.sync_copy(acc.at[pl.ds(r0, nr)], zbuf.at[pl.ds(0, nr)])
        pltpu.sync_copy(zbuf.at[pl.ds(0, nr)],
                        out_hbm.at[pl.ds(pl.multiple_of(c * N + r0, 8), nr)])

    _rows_copy(s, drain)


@functools.partial(
    pl.kernel,
    out_type=jax.ShapeDtypeStruct((NC, N, D), jnp.float32),
    mesh=_mesh,
    scratch_types=[
        [pltpu.VMEM((K,), jnp.int32)] * 6,
        [pltpu.VMEM((K,), jnp.int32)] * 6,
        [pltpu.VMEM((K, D), jnp.float32)] * 4,
        pltpu.VMEM_SHARED((N, D), jnp.float32),
        [pltpu.SemaphoreType.DMA] * 6,
        [pltpu.SemaphoreType.DMA] * 4,
        [pltpu.SemaphoreType.DMA] * 4,
    ],
)
def _agg_kernel(g_hbm, src_hbm, dst_hbm, zeros_hbm, out_hbm,
                sv, dv, rows, acc, semi, semg, sems):
    c = lax.axis_index("c")
    s = lax.axis_index("s")
    t = s * NC + c
    _rows_copy(s, lambda r0, nr: pltpu.sync_copy(
        zeros_hbm.at[pl.ds(r0, nr)], acc.at[pl.ds(r0, nr)]))
    base = pl.multiple_of(t * EC, 8)
    plsc.subcore_barrier()

    def load_idx(i, z):
        off = pl.multiple_of(base + i * K, 8)
        pltpu.async_copy(src_hbm.at[pl.ds(off, K)], sv[z], semi[z])
        pltpu.async_copy(dst_hbm.at[pl.ds(off, K)], dv[z], semi[z])

    def wait_idx(z):
        pltpu.make_async_copy(src_hbm.at[pl.ds(base, K)], sv[z], semi[z]).wait()
        pltpu.make_async_copy(dst_hbm.at[pl.ds(base, K)], dv[z], semi[z]).wait()

    def gather(z, x):
        pltpu.async_copy(g_hbm.at[sv[z]], rows[x], semg[x])

    def wait_gather(x):
        pltpu.make_async_copy(g_hbm.at[sv[0]], rows[x], semg[x]).wait()

    def scatter(x, w):
        pltpu.async_copy(rows[x], acc.at[dv[w]], sems[x], add=True)

    def wait_scatter(x):
        pltpu.make_async_copy(rows[x], acc.at[dv[0]], sems[x]).wait()

    def steady(i, m12):
        x = m12 % 4
        wait_scatter(x)
        load_idx(jnp.minimum(i + 2, NB - 1), (m12 + 2) % 6)
        wait_idx(m12 % 6)
        gather(m12 % 6, x)
        wait_gather((m12 + 2) % 4)
        scatter((m12 + 2) % 4, (m12 + 4) % 6)

    load_idx(0, 0)
    load_idx(1, 1)
    for i in (0, 1):
        wait_idx(i)
        gather(i, i)
        load_idx(i + 2, i + 2)
    for i in (2, 3):
        wait_idx(i)
        gather(i, i)
        load_idx(i + 2, i + 2)
        wait_gather(i - 2)
        scatter(i - 2, i - 2)

    def body(j, carry):
        i0 = 4 + 12 * j
        for u in range(12):
            steady(i0 + u, (4 + u) % 12)
        return carry

    q = (NB - 5) // 12
    lax.fori_loop(0, q, body, 0)
    for i in range(4 + 12 * q, NB):
        steady(i, i % 12)
    wait_gather((NB - 2) % 4)
    scatter((NB - 2) % 4, (NB - 2) % 6)
    wait_gather((NB - 1) % 4)
    scatter((NB - 1) % 4, (NB - 1) % 6)
    wait_idx(((NB - 2) % 12 + 2) % 6)
    wait_idx(((NB - 1) % 12 + 2) % 6)
    for x in range(4):
        wait_scatter(x)
    plsc.subcore_barrier()
    _rows_copy(s, lambda r0, nr: pltpu.sync_copy(
        acc.at[pl.ds(r0, nr)], out_hbm.at[c, pl.ds(r0, nr)]))


def _dinv(degp_ref):
    deg = 1.0 + degp_ref[:, 0:1] + degp_ref[:, 1:2]
    return lax.rsqrt(deg)


def _tc_first_body(degp_ref, x_ref, w_ref, o_ref):
    dinv = _dinv(degp_ref)
    o_ref[...] = jnp.dot(dinv * x_ref[...], w_ref[...],
                         preferred_element_type=jnp.float32)


def _tc_mid_body(degp_ref, p_ref, g_ref, b_ref, w_ref, o_ref):
    dinv = _dinv(degp_ref)
    pre = dinv * (p_ref[0] + p_ref[1] + g_ref[...]) + b_ref[...]
    h = jnp.maximum(pre, 0.0)
    o_ref[...] = jnp.dot(dinv * h, w_ref[...], preferred_element_type=jnp.float32)


def _tc_last_body(degp_ref, p_ref, g_ref, b_ref, wl_ref, bl_ref, o_ref):
    dinv = _dinv(degp_ref)
    pre = dinv * (p_ref[0] + p_ref[1] + g_ref[...]) + b_ref[...]
    o_ref[...] = jnp.dot(pre, wl_ref[...],
                         preferred_element_type=jnp.float32) + bl_ref[...]


_degp_spec = pl.BlockSpec((RB, NC), lambda i: (i, 0))
_p_spec = pl.BlockSpec((NC, RB, D), lambda i: (0, i, 0))
_row_spec = pl.BlockSpec((RB, D), lambda i: (i, 0))
_b_spec = pl.BlockSpec((1, D), lambda i: (0, 0))
_w_spec = pl.BlockSpec((D, D), lambda i: (0, 0))
_out_shape = jax.ShapeDtypeStruct((N, D), jnp.float32)
_grid = (N // RB,)

_tc_first = pl.pallas_call(
    _tc_first_body, grid=_grid,
    in_specs=[_degp_spec, _row_spec, _w_spec],
    out_specs=_row_spec, out_shape=_out_shape)

_tc_mid = pl.pallas_call(
    _tc_mid_body, grid=_grid,
    in_specs=[_degp_spec, _p_spec, _row_spec, _b_spec, _w_spec],
    out_specs=_row_spec, out_shape=_out_shape)

_tc_last = pl.pallas_call(
    _tc_last_body, grid=_grid,
    in_specs=[_degp_spec, _p_spec, _row_spec, _b_spec, _w_spec, _b_spec],
    out_specs=_row_spec, out_shape=_out_shape)


def kernel(x, adj_t, W0, b0, W1, b1, W2, b2, Wl, bl):
    src = adj_t[0]
    dst = adj_t[1]
    zeros128 = jnp.zeros((N, D), jnp.float32)

    degp = _deg_kernel(dst).reshape(NC, N).T
    g0 = _tc_first(degp, x, W0)
    p = _agg_kernel(g0, src, dst, zeros128)
    g1 = _tc_mid(degp, p, g0, b0.reshape(1, D), W1)
    p = _agg_kernel(g1, src, dst, zeros128)
    g2 = _tc_mid(degp, p, g1, b1.reshape(1, D), W2)
    p = _agg_kernel(g2, src, dst, zeros128)
    out = _tc_last(degp, p, g2, b2.reshape(1, D), Wl, bl.reshape(1, D))
    return out


# speedup vs baseline: 1.0356x; 1.0356x over previous
import functools

import jax
import jax.numpy as jnp
from jax import lax
from jax.experimental import pallas as pl
from jax.experimental.pallas import tpu as pltpu
from jax.experimental.pallas import tpu_sc as plsc

N, E, D = 10000, 320000, 128
NC, NS = 2, 16
NW = NC * NS
EC = E // NW
K = 80
NB = EC // K
RPT = 624
RLAST = N - RPT * (NS - 1)
DEGW = 16
RB = 2000

_mesh = plsc.VectorSubcoreMesh(core_axis_name="c", subcore_axis_name="s")


def _rows_copy(s, copy):
    @pl.when(s < NS - 1)
    def _():
        copy(pl.multiple_of(s * RPT, 8), RPT)

    @pl.when(s == NS - 1)
    def _():
        copy((NS - 1) * RPT, RLAST)


@functools.partial(
    pl.kernel,
    out_type=jax.ShapeDtypeStruct((NC * N,), jnp.float32),
    mesh=_mesh,
    scratch_types=[
        pltpu.VMEM((EC,), jnp.int32),
        [pltpu.VMEM((K,), jnp.int32)] * 4,
        pltpu.VMEM((K,), jnp.float32),
        pltpu.VMEM((RLAST,), jnp.float32),
        pltpu.VMEM_SHARED((N,), jnp.float32),
        [pltpu.SemaphoreType.DMA] * 4,
    ],
)
def _deg_kernel(dst_hbm, out_hbm, dstbig, dv, ones_v, zbuf, acc, sems):
    c = lax.axis_index("c")
    s = lax.axis_index("s")
    t = s * NC + c
    zeros = jnp.zeros((16,), jnp.float32)
    for j in range(RLAST // 16):
        zbuf[pl.ds(j * 16, 16)] = zeros
    _rows_copy(s, lambda r0, nr: pltpu.sync_copy(
        zbuf.at[pl.ds(0, nr)], acc.at[pl.ds(r0, nr)]))
    ones = jnp.full((16,), 1.0, jnp.float32)
    for j in range(K // 16):
        ones_v[pl.ds(j * 16, 16)] = ones
    pltpu.sync_copy(dst_hbm.at[pl.ds(pl.multiple_of(t * EC, 8), EC)], dstbig)
    plsc.subcore_barrier()

    def fill(i, x):
        for j in range(K // 16):
            dv[x][pl.ds(j * 16, 16)] = dstbig[pl.ds(i * K + j * 16, 16)]

    def scat(x):
        pltpu.async_copy(ones_v, acc.at[dv[x]], sems[x], add=True)

    def wait_scat(x):
        pltpu.make_async_copy(ones_v, acc.at[dv[x]], sems[x]).wait()

    for i in range(4):
        fill(i, i)
        scat(i)

    def body(j, carry):
        i0 = 4 + 4 * j
        for u in range(4):
            wait_scat(u)
            fill(i0 + u, u)
            scat(u)
        return carry

    lax.fori_loop(0, (NB - 5) // 4, body, 0)
    wait_scat(0)
    fill(NB - 1, 0)
    scat(0)
    for x in (1, 2, 3, 0):
        wait_scat(x)
    plsc.subcore_barrier()

    def drain(r0, nr):
        pltpu.sync_copy(acc.at[pl.ds(r0, nr)], zbuf.at[pl.ds(0, nr)])
        pltpu.sync_copy(zbuf.at[pl.ds(0, nr)],
                        out_hbm.at[pl.ds(pl.multiple_of(c * N + r0, 8), nr)])

    _rows_copy(s, drain)


@functools.partial(
    pl.kernel,
    out_type=jax.ShapeDtypeStruct((NC, N, D), jnp.float32),
    mesh=_mesh,
    scratch_types=[
        pltpu.VMEM((EC,), jnp.int32),
        [pltpu.VMEM((K,), jnp.int32)] * 3,
        [pltpu.VMEM((K, D), jnp.float32)] * 3,
        pltpu.VMEM_SHARED((N, D), jnp.float32),
        [pltpu.SemaphoreType.DMA] * 3,
        [pltpu.SemaphoreType.DMA] * 3,
        [pltpu.SemaphoreType.DMA] * 3,
    ],
)
def _agg_kernel(g_hbm, src_hbm, dst_hbm, zeros_hbm, out_hbm,
                srcbig, dv, rows, acc, semi, semg, sems):
    c = lax.axis_index("c")
    s = lax.axis_index("s")
    t = s * NC + c
    _rows_copy(s, lambda r0, nr: pltpu.sync_copy(
        zeros_hbm.at[pl.ds(r0, nr)], acc.at[pl.ds(r0, nr)]))
    base = pl.multiple_of(t * EC, 8)
    pltpu.sync_copy(src_hbm.at[pl.ds(base, EC)], srcbig)
    plsc.subcore_barrier()

    def start(i, x):
        off = pl.multiple_of(base + i * K, 8)
        pltpu.async_copy(dst_hbm.at[pl.ds(off, K)], dv[x], semi[x])
        pltpu.async_copy(g_hbm.at[srcbig.at[pl.ds(i * K, K)]], rows[x], semg[x])

    def wait_gather(x):
        pltpu.make_async_copy(
            g_hbm.at[srcbig.at[pl.ds(0, K)]], rows[x], semg[x]).wait()
        pltpu.make_async_copy(dst_hbm.at[pl.ds(base, K)], dv[x], semi[x]).wait()

    def scatter(x):
        pltpu.async_copy(rows[x], acc.at[dv[x]], sems[x], add=True)

    def wait_scatter(x):
        pltpu.make_async_copy(rows[x], acc.at[dv[x]], sems[x]).wait()

    def steady(i, x):
        wait_scatter(x)
        start(i, x)
        y = (x + 2) % 3
        wait_gather(y)
        scatter(y)

    start(0, 0)
    start(1, 1)
    wait_gather(0)
    scatter(0)
    start(2, 2)
    wait_gather(1)
    scatter(1)

    def body(j, carry):
        i0 = 3 + 3 * j
        for u in range(3):
            steady(i0 + u, u)
        return carry

    q = (NB - 3) // 3
    lax.fori_loop(0, q, body, 0)
    for i in range(3 + 3 * q, NB):
        steady(i, i % 3)
    y = (NB - 1) % 3
    wait_gather(y)
    scatter(y)
    for x in range(3):
        wait_scatter(x)
    plsc.subcore_barrier()
    _rows_copy(s, lambda r0, nr: pltpu.sync_copy(
        acc.at[pl.ds(r0, nr)], out_hbm.at[c, pl.ds(r0, nr)]))


def _dinv(degp_ref):
    deg = 1.0 + degp_ref[:, 0:1] + degp_ref[:, 1:2]
    return lax.rsqrt(deg)


def _tc_first_body(degp_ref, x_ref, w_ref, o_ref):
    dinv = _dinv(degp_ref)
    o_ref[...] = jnp.dot(dinv * x_ref[...], w_ref[...],
                         preferred_element_type=jnp.float32)


def _tc_mid_body(degp_ref, p_ref, g_ref, b_ref, w_ref, o_ref):
    dinv = _dinv(degp_ref)
    pre = dinv * (p_ref[0] + p_ref[1] + g_ref[...]) + b_ref[...]
    h = jnp.maximum(pre, 0.0)
    o_ref[...] = jnp.dot(dinv * h, w_ref[...], preferred_element_type=jnp.float32)


def _tc_last_body(degp_ref, p_ref, g_ref, b_ref, wl_ref, bl_ref, o_ref):
    dinv = _dinv(degp_ref)
    pre = dinv * (p_ref[0] + p_ref[1] + g_ref[...]) + b_ref[...]
    o_ref[...] = jnp.dot(pre, wl_ref[...],
                         preferred_element_type=jnp.float32) + bl_ref[...]


_degp_spec = pl.BlockSpec((RB, NC), lambda i: (i, 0))
_p_spec = pl.BlockSpec((NC, RB, D), lambda i: (0, i, 0))
_row_spec = pl.BlockSpec((RB, D), lambda i: (i, 0))
_b_spec = pl.BlockSpec((1, D), lambda i: (0, 0))
_w_spec = pl.BlockSpec((D, D), lambda i: (0, 0))
_out_shape = jax.ShapeDtypeStruct((N, D), jnp.float32)
_grid = (N // RB,)

_tc_first = pl.pallas_call(
    _tc_first_body, grid=_grid,
    in_specs=[_degp_spec, _row_spec, _w_spec],
    out_specs=_row_spec, out_shape=_out_shape)

_tc_mid = pl.pallas_call(
    _tc_mid_body, grid=_grid,
    in_specs=[_degp_spec, _p_spec, _row_spec, _b_spec, _w_spec],
    out_specs=_row_spec, out_shape=_out_shape)

_tc_last = pl.pallas_call(
    _tc_last_body, grid=_grid,
    in_specs=[_degp_spec, _p_spec, _row_spec, _b_spec, _w_spec, _b_spec],
    out_specs=_row_spec, out_shape=_out_shape)


def kernel(x, adj_t, W0, b0, W1, b1, W2, b2, Wl, bl):
    src = adj_t[0]
    dst = adj_t[1]
    zeros128 = jnp.zeros((N, D), jnp.float32)

    degp = _deg_kernel(dst).reshape(NC, N).T
    g0 = _tc_first(degp, x, W0)
    p = _agg_kernel(g0, src, dst, zeros128)
    g1 = _tc_mid(degp, p, g0, b0.reshape(1, D), W1)
    p = _agg_kernel(g1, src, dst, zeros128)
    g2 = _tc_mid(degp, p, g1, b1.reshape(1, D), W2)
    p = _agg_kernel(g2, src, dst, zeros128)
    out = _tc_last(degp, p, g2, b2.reshape(1, D), Wl, bl.reshape(1, D))
    return out
